# Initial kernel scaffold; baseline (speedup 1.0000x reference)
import jax
import jax.numpy as jnp
from jax.experimental import pallas as pl


def kernel(relative_mat, embedding):
    raise NotImplementedError("write your pallas kernel here")


# speedup vs baseline: 8.2349x; 8.2349x over previous
import functools

import jax
import jax.numpy as jnp
from jax.experimental import pallas as pl
from jax.experimental.pallas import tpu as pltpu

_ROWS = 2048
_COLS = 2048
_UNITS = 64
_NLEVELS = 5

_BI = 256
_W = 128
_BW = _W * _UNITS


def _gather_kernel(idx_ref, rep_ref, embt_ref, out_ref):
    idxf = idx_ref[...].astype(jnp.bfloat16)
    s = jnp.dot(idxf, rep_ref[...], preferred_element_type=jnp.float32)
    e = embt_ref[...]
    acc = jnp.broadcast_to(e[0:1, :], s.shape)
    for k in range(1, _NLEVELS):
        acc = jnp.where(s >= (k - 0.5), jnp.broadcast_to(e[k : k + 1, :], s.shape), acc)
    out_ref[...] = acc


@functools.partial(jax.jit, static_argnames=())
def _run(relative_mat, embedding):
    n_i = _ROWS // _BI
    n_j = _COLS // _W

    q = jnp.arange(_W, dtype=jnp.int32)[:, None]
    c = jnp.arange(_BW, dtype=jnp.int32)[None, :]
    rep = (c // _UNITS == q).astype(jnp.bfloat16)

    embt = jnp.tile(embedding, (1, _W))

    out2d = pl.pallas_call(
        _gather_kernel,
        grid=(n_i, n_j),
        in_specs=[
            pl.BlockSpec((_BI, _W), lambda i, j: (i, j)),
            pl.BlockSpec((_W, _BW), lambda i, j: (0, 0)),
            pl.BlockSpec((_NLEVELS, _BW), lambda i, j: (0, 0)),
        ],
        out_specs=pl.BlockSpec((_BI, _BW), lambda i, j: (i, j)),
        out_shape=jax.ShapeDtypeStruct((_ROWS, _COLS * _UNITS), jnp.float32),
        compiler_params=pltpu.CompilerParams(
            dimension_semantics=("parallel", "arbitrary"),
        ),
    )(relative_mat, rep, embt)
    return out2d.reshape(_ROWS, _COLS, _UNITS)


def kernel(relative_mat, embedding):
    return _run(relative_mat, embedding)
